# baseline (speedup 1.0000x reference)
import functools

import numpy as np
import jax
import jax.numpy as jnp
from jax.experimental import pallas as pl
from jax.experimental.pallas import tpu as pltpu

_D = 18
_GS = 9
_NA = 16
_NB = 32
_NSMALL = 2 * (_NA + _NB)
_SQRT_D = np.float32(np.sqrt(np.float32(18.0)))
_QS = np.float32(np.float32(1.0) / _SQRT_D)
_HALF_QS = np.float32(_QS / np.float32(2.0))
_C2 = np.float32(np.float32(2.0) * _QS)
_ROWS = 8192
_LANES = 128


def _pats(nbits):
    codes = np.arange(1 << nbits)
    gb = 2 ** np.arange(nbits - 1, -1, -1)
    return (((codes[:, None] // gb) % 2) * 2 - 1).astype(np.float32)


def _codebook():
    cba = _pats(4)
    cbb = _pats(5)
    w = np.zeros((_NSMALL, _D), np.float32)
    w[0:16, 0:4] = cba
    w[16:48, 4:9] = cbb
    w[48:64, 9:13] = cba
    w[64:96, 13:18] = cbb
    return jnp.asarray(w)


def _basis():
    w = np.zeros((3, _D), np.float32)
    w[0] = 2.0 ** np.arange(-1, -1 - _D, -1)
    w[1, :_GS] = 2.0 ** np.arange(-1, -1 - _GS, -1)
    w[2, _GS:] = 2.0 ** np.arange(-1, -1 - _GS, -1)
    return jnp.asarray(w)


def _sum_pattern_t():
    p = np.zeros((4, _NSMALL), np.float32)
    p[0, 0:16] = 1.0
    p[1, 16:48] = 1.0
    p[2, 48:64] = 1.0
    p[3, 64:96] = 1.0
    return jnp.asarray(p)


def _bsq_kernel(zt_ref, zf_ref, wc_ref, wb_ref, p_ref,
                zq_ref, idx3_ref, avgp_ref, loss_ref, cbe_ref,
                acc_ref, s_ref, *, ntot):
    pid = pl.program_id(0)
    nsteps = pl.num_programs(0)

    @pl.when(pid == 0)
    def _init():
        acc_ref[...] = jnp.zeros_like(acc_ref)
        s_ref[0] = jnp.float32(0.0)
        s_ref[1] = jnp.float32(0.0)

    zf = zf_ref[...]
    zq_ref[...] = jnp.where(zf > 0, _QS, -_QS)

    zt = zt_ref[...]
    zhat = jnp.where(zt > 0, jnp.float32(1.0), jnp.float32(-1.0))
    diff = zhat * _QS - zt
    s_ref[1] += jnp.sum(diff * diff)

    b3 = jax.lax.dot_general(wb_ref[...], zhat, (((1,), (0,)), ((), ())),
                             preferred_element_type=jnp.float32)
    idx_f = 131071.5 + (_HALF_QS * 262144.0) * b3[0:1, :]
    g = 255.5 + (_HALF_QS * 512.0) * b3[1:3, :]
    idx3_ref[...] = jnp.concatenate([idx_f, g], axis=0).astype(jnp.int32)

    o1 = jax.lax.dot_general(wc_ref[...], zt, (((1,), (0,)), ((), ())),
                             preferred_element_type=jnp.float32)
    lgt = o1 * _C2
    et = jnp.exp(lgt)
    elt = et * lgt
    s4 = jax.lax.dot_general(p_ref[...], et, (((1,), (0,)), ((), ())),
                             preferred_element_type=jnp.float32)
    t4 = jax.lax.dot_general(p_ref[...], elt, (((1,), (0,)), ((), ())),
                             preferred_element_type=jnp.float32)
    ra = 1.0 / s4
    s_ref[0] += jnp.sum(jnp.log(s4)) - jnp.sum(t4 * ra)
    r0 = ra[0:1, :] * ra[1:2, :]
    r1 = ra[2:3, :] * ra[3:4, :]
    a0 = jax.lax.dot_general(et[:_NA, :] * r0, et[_NA:_NA + _NB, :],
                             (((1,), (1,)), ((), ())),
                             preferred_element_type=jnp.float32)
    a1 = jax.lax.dot_general(et[48:48 + _NA, :] * r1, et[64:, :],
                             (((1,), (1,)), ((), ())),
                             preferred_element_type=jnp.float32)
    acc_ref[:, :_NB] += a0
    acc_ref[:, _NB:] += a1

    @pl.when(pid == nsteps - 1)
    def _fin():
        inv_n = jnp.float32(1.0 / ntot)
        acc = acc_ref[...] * inv_n
        avgp_ref[...] = acc
        cbe = -jnp.sum(acc * jnp.log(acc + 1e-8))
        cbe_ref[...] = jnp.reshape(cbe, (1, 1))
        pse = s_ref[0] * inv_n
        commit = 0.25 * (s_ref[1] * inv_n)
        loss_ref[...] = jnp.reshape(commit + pse - cbe, (1, 1))


def kernel(z):
    b, s, d = z.shape
    n = b * s
    zt = z.reshape(n, d).T
    fl_rows = n * d // _LANES
    zflat = z.reshape(fl_rows, _LANES)
    fl_blk = _ROWS * d // _LANES
    wc = _codebook()
    wb = _basis()
    pat = _sum_pattern_t()
    grid = n // _ROWS
    outs = pl.pallas_call(
        functools.partial(_bsq_kernel, ntot=float(n)),
        grid=(grid,),
        in_specs=[
            pl.BlockSpec((d, _ROWS), lambda i: (0, i)),
            pl.BlockSpec((fl_blk, _LANES), lambda i: (i, 0)),
            pl.BlockSpec((_NSMALL, d), lambda i: (0, 0)),
            pl.BlockSpec((3, d), lambda i: (0, 0)),
            pl.BlockSpec((4, _NSMALL), lambda i: (0, 0)),
        ],
        out_specs=[
            pl.BlockSpec((fl_blk, _LANES), lambda i: (i, 0)),
            pl.BlockSpec((3, _ROWS), lambda i: (0, i)),
            pl.BlockSpec((_NA, 2 * _NB), lambda i: (0, 0)),
            pl.BlockSpec((1, 1), lambda i: (0, 0)),
            pl.BlockSpec((1, 1), lambda i: (0, 0)),
        ],
        out_shape=[
            jax.ShapeDtypeStruct((fl_rows, _LANES), jnp.float32),
            jax.ShapeDtypeStruct((3, n), jnp.int32),
            jax.ShapeDtypeStruct((_NA, 2 * _NB), jnp.float32),
            jax.ShapeDtypeStruct((1, 1), jnp.float32),
            jax.ShapeDtypeStruct((1, 1), jnp.float32),
        ],
        scratch_shapes=[
            pltpu.VMEM((_NA, 2 * _NB), jnp.float32),
            pltpu.SMEM((2,), jnp.float32),
        ],
        compiler_params=pltpu.CompilerParams(
            dimension_semantics=("arbitrary",)),
    )(zt, zflat, wc, wb, pat)
    zqf, idx3, avgp_raw, loss, cbe = outs
    zq = zqf.reshape(b, s, d)
    indices = idx3[0].reshape(b, s).astype(jnp.int64)
    group_indices = idx3[1:3].T.reshape(b, s, 2).astype(jnp.int64)
    avgp = jnp.stack([avgp_raw[:, :_NB].reshape(_NA * _NB),
                      avgp_raw[:, _NB:].reshape(_NA * _NB)], axis=0)
    return (zq, loss[0, 0], cbe[0, 0], indices, group_indices, avgp)
